# baseline (speedup 1.0000x reference)
import functools

import jax
import jax.numpy as jnp
from jax import lax
from jax.experimental import pallas as pl
from jax.experimental.pallas import tpu as pltpu
from jax.experimental.pallas import tpu_sc as plsc

B = 4096
L = 20
V = 1000
ROW = L * V

_info = plsc.get_sparse_core_info()
NC, NS, LANES = _info.num_cores, _info.num_subcores, _info.num_lanes
NW = NC * NS
RPW = B // NW


def _row_indices(xv, r):
    lane = lax.iota(jnp.int32, LANES)
    i0 = xv[pl.ds(r * L, LANES)] + lane * V
    i1 = xv[pl.ds(r * L + 4, LANES)] + (lane + 4) * V
    m1 = lane >= (LANES - 4)
    return i0, i1, m1


def _scatter_row(buf, xv, r, val):
    i0, i1, m1 = _row_indices(xv, r)
    vvec = jnp.full((LANES,), val, jnp.float32)
    plsc.store_scatter(buf, [i0], vvec)
    plsc.store_scatter(buf, [i1], vvec, mask=m1)


def _onehot_body(x_hbm, out_hbm, xv, buf0, buf1, sem0, sem1):
    bufs = (buf0, buf1)
    sems = (sem0, sem1)
    wid = lax.axis_index("s") * NC + lax.axis_index("c")
    base = wid * RPW

    pltpu.sync_copy(x_hbm.at[pl.ds(base * L, RPW * L)], xv)

    zvec = jnp.zeros((LANES,), jnp.float32)

    def zero_body(i, _):
        buf0[pl.ds(i * LANES, LANES)] = zvec
        buf1[pl.ds(i * LANES, LANES)] = zvec
        return 0

    lax.fori_loop(0, ROW // LANES, zero_body, 0)

    for b in range(2):
        _scatter_row(bufs[b], xv, b, 1.0)
        pltpu.make_async_copy(bufs[b], out_hbm.at[base + b], sems[b]).start()

    def pair_body(g, _):
        for b in range(2):
            r = 2 * g + b
            prev = r - 2
            pltpu.make_async_copy(
                bufs[b], out_hbm.at[base + prev], sems[b]
            ).wait()
            _scatter_row(bufs[b], xv, prev, 0.0)
            _scatter_row(bufs[b], xv, r, 1.0)
            pltpu.make_async_copy(
                bufs[b], out_hbm.at[base + r], sems[b]
            ).start()
        return 0

    lax.fori_loop(1, RPW // 2, pair_body, 0)

    for b in range(2):
        pltpu.make_async_copy(
            bufs[b], out_hbm.at[base + RPW - 2 + b], sems[b]
        ).wait()


@jax.jit
def _onehot(x_flat):
    mesh = plsc.VectorSubcoreMesh(core_axis_name="c", subcore_axis_name="s")
    f = functools.partial(
        pl.kernel,
        out_type=jax.ShapeDtypeStruct((B, ROW), jnp.float32),
        mesh=mesh,
        scratch_types=[
            pltpu.VMEM((RPW * L,), jnp.int32),
            pltpu.VMEM((ROW,), jnp.float32),
            pltpu.VMEM((ROW,), jnp.float32),
            pltpu.SemaphoreType.DMA,
            pltpu.SemaphoreType.DMA,
        ],
        compiler_params=pltpu.CompilerParams(needs_layout_passes=False),
    )(_onehot_body)
    return f(x_flat)


def kernel(x):
    out = _onehot(x.reshape(B * L))
    return out.reshape(B, L, V)
